# baseline (speedup 1.0000x reference)
import jax
import jax.numpy as jnp
from jax.experimental import pallas as pl
from jax.experimental.pallas import tpu as pltpu

_F32 = jnp.float32
_BF16 = jnp.bfloat16

_RS2 = ((1, 0), (0, 1), (1, 1), (0, 2))
_RD1 = {0: ((0, 0), (1, 2)), 1: ((1, 1), (2, 3))}
_RD2 = (
    ((1, 0, 0), (0, 1, 2)),
    ((0, 1, 1), (1, 1, 3)),
    ((0, 1, 0), (1, 1, 2)),
    ((1, 1, 1), (0, 2, 3)),
)


def _fold8(a):
    a = a[0:64] + a[64:128]
    a = a[0:32] + a[32:64]
    a = a[0:16] + a[16:32]
    return a[0:8] + a[8:16]


def _f32(a):
    return a.astype(_F32)


def _conv1_chan(xq_ref, w1_ref, pi, pj, o):
    acc = None
    for di in range(4):
        m = di + 2 * pi
        g, k0 = m & 3, m >> 2
        for dj in range(4):
            n = dj + 2 * pj
            g2, l0 = n & 3, n >> 2
            xs = _f32(xq_ref[0, g, g2, k0:k0 + 128, l0:l0 + 128])
            term = _f32(w1_ref[di * 4 + dj])[o] * xs
            acc = term if acc is None else acc + term
    return acc


def _deconv1_parities(dpad_ref, w1d_ref):
    for pi in (0, 1):
        for pj in (0, 1):
            acc = None
            for (sr, ki) in _RD1[pi]:
                for (sc, kj) in _RD1[pj]:
                    ds = _f32(dpad_ref[:, sr:sr + 128, sc:sc + 128])
                    for c in range(4):
                        term = (_f32(w1d_ref[c, ki * 4 + kj])[:, None, None]
                                * ds[c][None, :, :])
                        acc = term if acc is None else acc + term
            yield pi, pj, acc


def _k1_body(xq_ref, w1_ref, st_ref):
    for o in range(16):
        s = None
        sq = None
        for pi in (0, 1):
            for pj in (0, 1):
                acc = _conv1_chan(xq_ref, w1_ref, pi, pj, o)
                t1, t2 = _fold8(acc), _fold8(acc * acc)
                s = t1 if s is None else s + t1
                sq = t2 if sq is None else sq + t2
        st_ref[0, 0, o] = s
        st_ref[0, 1, o] = sq


def _k2_body(xq_ref, w1c_ref, aff1_ref, w2_ref, h2_ref, st_ref, hpad_ref):
    hpad_ref[...] = jnp.zeros((2, 2, 16, 130, 130), _BF16)

    def conv1_o(o, _):
        for pi in (0, 1):
            for pj in (0, 1):
                acc = None
                for di in range(4):
                    m = di + 2 * pi
                    g, k0 = m & 3, m >> 2
                    for dj in range(4):
                        n = dj + 2 * pj
                        g2, l0 = n & 3, n >> 2
                        xs = _f32(xq_ref[0, g, g2, k0:k0 + 128, l0:l0 + 128])
                        term = _f32(w1c_ref[o, 0])[di * 4 + dj] * xs
                        acc = term if acc is None else acc + term
                h = jnp.maximum(
                    (acc - aff1_ref[o, 0, 0]) / aff1_ref[o, 0, 1]
                    * aff1_ref[o, 0, 2] + aff1_ref[o, 0, 3], 0.0)
                hpad_ref[pi, pj, o, 1:129, 1:129] = h.astype(_BF16)
        return 0

    jax.lax.fori_loop(0, 16, conv1_o, 0)

    def conv2_c(c, accs):
        new = list(accs)
        for di in range(4):
            pi, sr = _RS2[di]
            for dj in range(4):
                pj, sc = _RS2[dj]
                t = di * 4 + dj
                hs = _f32(hpad_ref[pi, pj, c, sr:sr + 128, sc:sc + 128])
                for d in range(4):
                    new[d] = new[d] + _f32(w2_ref[c, t])[d] * hs
        return tuple(new)

    z = jnp.zeros((128, 128), _F32)
    accs = jax.lax.fori_loop(0, 16, conv2_c, (z, z, z, z))
    h2_ref[0] = jnp.stack(accs)
    for d in range(4):
        st_ref[0, 0, d] = _fold8(accs[d])
        st_ref[0, 1, d] = _fold8(accs[d] * accs[d])


def _k3_body(h2_ref, aff2_ref, pq_ref, pqb_ref, cb_ref, cbb_ref, poq_ref,
             pob_ref, w1d_ref, d0_ref, st_ref, dpad_ref):
    mv = aff2_ref[0][:, None, None]
    sv = aff2_ref[1][:, None, None]
    gv = aff2_ref[2][:, None, None]
    bv = aff2_ref[3][:, None, None]
    hq = jnp.maximum((h2_ref[0] - mv) / sv * gv + bv, 0.0)
    hqb = hq.astype(_BF16).astype(_F32)
    pq = _f32(pq_ref[...])
    q0 = (pq[0, 0] * hqb[0] + pq[0, 1] * hqb[1]
          + pq[0, 2] * hqb[2] + pq[0, 3] * hqb[3] + pqb_ref[0, 0])
    q1 = (pq[1, 0] * hqb[0] + pq[1, 1] * hqb[1]
          + pq[1, 2] * hqb[2] + pq[1, 3] * hqb[3] + pqb_ref[0, 1])
    q0b = q0.astype(_BF16).astype(_F32)
    q1b = q1.astype(_BF16).astype(_F32)
    qsq = q0 * q0 + q1 * q1
    cb0v = _f32(cbb_ref[0])
    cb1v = _f32(cbb_ref[1])
    best = jnp.full((128, 128), jnp.inf, _F32)
    bq0 = jnp.zeros((128, 128), _F32)
    bq1 = jnp.zeros((128, 128), _F32)
    for k in range(64):
        ck0 = cb_ref[0, k]
        ck1 = cb_ref[1, k]
        ck0b = cb0v[k]
        ck1b = cb1v[k]
        d = (qsq + (ck0 * ck0 + ck1 * ck1)) - 2.0 * (q0b * ck0b + q1b * ck1b)
        m = d < best
        best = jnp.where(m, d, best)
        bq0 = jnp.where(m, ck0, bq0)
        bq1 = jnp.where(m, ck1, bq1)
    bq0r = bq0.astype(_BF16).astype(_F32)
    bq1r = bq1.astype(_BF16).astype(_F32)
    lp = _fold8((bq0 - q0) ** 2 + (bq1 - q1) ** 2)
    poq = _f32(poq_ref[...])
    d0 = jnp.stack([poq[c, 0] * bq0r + poq[c, 1] * bq1r
                    + pob_ref[0, c] for c in range(4)])
    d0_ref[0] = d0
    dpad_ref[...] = jnp.zeros((4, 130, 130), _BF16)
    dpad_ref[:, 1:129, 1:129] = d0.astype(_BF16)
    s = [None] * 16
    sq = [None] * 16
    for _, _, acc in _deconv1_parities(dpad_ref, w1d_ref):
        for o in range(16):
            ao = acc[o]
            t1, t2 = _fold8(ao), _fold8(ao * ao)
            s[o] = t1 if s[o] is None else s[o] + t1
            sq[o] = t2 if sq[o] is None else sq[o] + t2
    for o in range(16):
        st_ref[0, 0, o] = s[o]
        st_ref[0, 1, o] = sq[o]
    st_ref[0, 2, 0] = lp


def _k4_body(d0_ref, xq_ref, aff3_ref, w1d_ref, w2d_ref, b2d_ref,
             og_ref, st_ref, dpad_ref, h3_ref):
    dpad_ref[...] = jnp.zeros((4, 130, 130), _BF16)
    dpad_ref[:, 1:129, 1:129] = d0_ref[0].astype(_BF16)
    h3_ref[...] = jnp.zeros((2, 2, 16, 130, 130), _BF16)
    a3 = aff3_ref[0][:, None, None]
    c3 = aff3_ref[1][:, None, None]
    for pi, pj, acc in _deconv1_parities(dpad_ref, w1d_ref):
        h3 = jnp.maximum(acc * a3 + c3, 0.0)
        h3_ref[pi, pj, :, 1:129, 1:129] = h3.astype(_BF16)
    rl = jnp.zeros((8, 128), _F32)
    b2 = b2d_ref[0, 0]
    for gr in range(4):
        for gc in range(4):
            o = None
            for (pi, sr, ki) in _RD2[gr]:
                for (pj, sc, kj) in _RD2[gc]:
                    hs = _f32(h3_ref[pi, pj, :, sr:sr + 128, sc:sc + 128])
                    for c in range(16):
                        term = _f32(w2d_ref[c])[ki * 4 + kj] * hs[c]
                        o = term if o is None else o + term
            o = jnp.tanh(o + b2)
            og_ref[0, gr, gc] = o
            xs = xq_ref[0, (gr + 1) & 3, (gc + 1) & 3,
                        (gr + 1) >> 2:((gr + 1) >> 2) + 128,
                        (gc + 1) >> 2:((gc + 1) >> 2) + 128]
            rl = rl + _fold8((xs - o) ** 2)
    st_ref[0, 0, 0] = rl


def _rep(shape):
    nd = len(shape)
    return pl.BlockSpec(shape, lambda i, _nd=nd: (0,) * _nd)


def _bf(a):
    return a.astype(_BF16).astype(_F32)


def kernel(x, conv1_w, conv1_b, bn1_g, bn1_b, conv2_w, conv2_b, bn2_g, bn2_b,
           preq_w, preq_b, codebook, postq_w, postq_b, dec1_w, dec1_b,
           bn3_g, bn3_b, dec2_w, dec2_b):
    B = x.shape[0]
    xp = jnp.pad(x[:, 0], ((0, 0), (1, 3), (1, 3)))
    xq4 = xp.reshape(B, 129, 4, 129, 4).transpose(0, 2, 4, 1, 3)
    xq4b = xq4.astype(_BF16)
    w1r = conv1_w[:, 0].reshape(16, 16).T.astype(_BF16)
    w1c = conv1_w[:, 0].reshape(16, 16)[:, None, :].astype(_BF16)
    w2r = conv2_w.transpose(1, 2, 3, 0).reshape(16, 16, 4).astype(_BF16)
    w1d = dec1_w.transpose(1, 2, 3, 0).reshape(4, 16, 16).astype(_BF16)
    w2d = dec2_w[0].reshape(16, 16).astype(_BF16)
    cbT = codebook.T
    cbB = codebook.T.astype(_BF16)
    pqm = preq_w[:, :, 0, 0].astype(_BF16)
    pqb = preq_b[None, :]
    pom = postq_w[:, :, 0, 0].astype(_BF16)
    pob = postq_b[None, :]
    b2d = dec2_b[None, :]

    st1 = pl.pallas_call(
        _k1_body,
        grid=(B,),
        in_specs=[
            pl.BlockSpec((1, 4, 4, 129, 129), lambda i: (i, 0, 0, 0, 0)),
            _rep((16, 16)),
        ],
        out_specs=pl.BlockSpec((1, 2, 16, 8, 128), lambda i: (i, 0, 0, 0, 0)),
        out_shape=jax.ShapeDtypeStruct((B, 2, 16, 8, 128), _F32),
    )(xq4b, w1r)
    n1 = B * 256 * 256
    m1 = jnp.sum(st1[:, 0], axis=(0, 2, 3)) / n1
    v1 = jnp.sum(st1[:, 1], axis=(0, 2, 3)) / n1 - m1 * m1
    s1 = jnp.sqrt(v1 + 1e-5)
    aff1 = jnp.stack([m1, s1, bn1_g, bn1_b], axis=-1)[:, None, :]

    h2, st2 = pl.pallas_call(
        _k2_body,
        grid=(B,),
        in_specs=[
            pl.BlockSpec((1, 4, 4, 129, 129), lambda i: (i, 0, 0, 0, 0)),
            _rep((16, 1, 16)),
            _rep((16, 1, 4)),
            _rep((16, 16, 4)),
        ],
        out_specs=[
            pl.BlockSpec((1, 4, 128, 128), lambda i: (i, 0, 0, 0)),
            pl.BlockSpec((1, 2, 4, 8, 128), lambda i: (i, 0, 0, 0, 0)),
        ],
        out_shape=[
            jax.ShapeDtypeStruct((B, 4, 128, 128), _F32),
            jax.ShapeDtypeStruct((B, 2, 4, 8, 128), _F32),
        ],
        scratch_shapes=[pltpu.VMEM((2, 2, 16, 130, 130), _BF16)],
    )(xq4b, w1c, aff1, w2r)
    n2 = B * 128 * 128
    m2 = jnp.sum(st2[:, 0], axis=(0, 2, 3)) / n2
    v2 = jnp.sum(st2[:, 1], axis=(0, 2, 3)) / n2 - m2 * m2
    aff2 = jnp.stack([m2, jnp.sqrt(v2 + 1e-5), bn2_g, bn2_b])

    d0, st3 = pl.pallas_call(
        _k3_body,
        grid=(B,),
        in_specs=[
            pl.BlockSpec((1, 4, 128, 128), lambda i: (i, 0, 0, 0)),
            _rep((4, 4)),
            _rep((2, 4)),
            _rep((1, 2)),
            _rep((2, 64)),
            _rep((2, 64)),
            _rep((4, 2)),
            _rep((1, 4)),
            _rep((4, 16, 16)),
        ],
        out_specs=[
            pl.BlockSpec((1, 4, 128, 128), lambda i: (i, 0, 0, 0)),
            pl.BlockSpec((1, 3, 16, 8, 128), lambda i: (i, 0, 0, 0, 0)),
        ],
        out_shape=[
            jax.ShapeDtypeStruct((B, 4, 128, 128), _F32),
            jax.ShapeDtypeStruct((B, 3, 16, 8, 128), _F32),
        ],
        scratch_shapes=[pltpu.VMEM((4, 130, 130), _BF16)],
    )(h2, aff2, pqm, pqb, cbT, cbB, pom, pob, w1d)
    n3 = B * 256 * 256
    m3 = jnp.sum(st3[:, 0], axis=(0, 2, 3)) / n3
    v3 = jnp.sum(st3[:, 1], axis=(0, 2, 3)) / n3 - m3 * m3
    a3 = bn3_g / jnp.sqrt(v3 + 1e-5)
    aff3 = jnp.stack([a3, bn3_b - m3 * a3])
    vq_sum = jnp.sum(st3[:, 2, 0])

    og, st4 = pl.pallas_call(
        _k4_body,
        grid=(B,),
        in_specs=[
            pl.BlockSpec((1, 4, 128, 128), lambda i: (i, 0, 0, 0)),
            pl.BlockSpec((1, 4, 4, 129, 129), lambda i: (i, 0, 0, 0, 0)),
            _rep((2, 16)),
            _rep((4, 16, 16)),
            _rep((16, 16)),
            _rep((1, 1)),
        ],
        out_specs=[
            pl.BlockSpec((1, 4, 4, 128, 128), lambda i: (i, 0, 0, 0, 0)),
            pl.BlockSpec((1, 1, 1, 8, 128), lambda i: (i, 0, 0, 0, 0)),
        ],
        out_shape=[
            jax.ShapeDtypeStruct((B, 4, 4, 128, 128), _F32),
            jax.ShapeDtypeStruct((B, 1, 1, 8, 128), _F32),
        ],
        scratch_shapes=[
            pltpu.VMEM((4, 130, 130), _BF16),
            pltpu.VMEM((2, 2, 16, 130, 130), _BF16),
        ],
    )(d0, xq4, aff3, w1d, w2d, b2d)

    out = og.transpose(0, 3, 1, 4, 2).reshape(B, 1, 512, 512)
    recon = jnp.sum(st4[:, 0, 0]) / (B * 512 * 512)
    qloss = 1.2 * (vq_sum / (B * 16384 * 2)) + recon
    return out, qloss
